# Initial kernel scaffold; baseline (speedup 1.0000x reference)
import jax
import jax.numpy as jnp
from jax.experimental import pallas as pl


def kernel(x, edge_index, batch_num_nodes, W_gat, a_src, a_dst, b_gat, Wq, Wk, Wv, Wo, bq, bk, bv, bo, ln1_g, ln1_b, ln2_g, ln2_b, Wff1, bff1, Wff2, bff2):
    raise NotImplementedError("write your pallas kernel here")


# speedup vs baseline: 1.2028x; 1.2028x over previous
import functools
import math

import jax
import jax.numpy as jnp
from jax import lax
from jax.experimental import pallas as pl
from jax.experimental.pallas import tpu as pltpu

N_NODES = 10000
F = 128
NB = 16
SEG = 625
NL = 2
NH = 4
DH = F // NH
DFF = 2048
DFF_CHUNK = 512


def _ln(h, g, b, eps=1e-5):
    mu = jnp.mean(h, axis=-1, keepdims=True)
    var = jnp.mean((h - mu) ** 2, axis=-1, keepdims=True)
    return (h - mu) * jax.lax.rsqrt(var + eps) * g + b


def _transformer_body(hb_ref, Wq_ref, Wk_ref, Wv_ref, Wo_ref, bq_ref, bk_ref,
                      bv_ref, bo_ref, ln1_g_ref, ln1_b_ref, ln2_g_ref,
                      ln2_b_ref, Wff1_ref, bff1_ref, Wff2_ref, bff2_ref,
                      out_ref):
    h = hb_ref[0]
    dots = functools.partial(lax.dot_general,
                             dimension_numbers=(((1,), (0,)), ((), ())),
                             preferred_element_type=jnp.float32)
    dots_t = functools.partial(lax.dot_general,
                               dimension_numbers=(((1,), (1,)), ((), ())),
                               preferred_element_type=jnp.float32)
    inv_sqrt_dh = 1.0 / math.sqrt(DH)

    l = 0
    q = dots(h, Wq_ref[l]) + bq_ref[l]
    k = dots(h, Wk_ref[l]) + bk_ref[l]
    v = dots(h, Wv_ref[l]) + bv_ref[l]
    o_heads = []
    for i in range(NH):
        qi = q[:, i * DH:(i + 1) * DH]
        ki = k[:, i * DH:(i + 1) * DH]
        vi = v[:, i * DH:(i + 1) * DH]
        sc = dots_t(qi, ki) * inv_sqrt_dh
        m = jnp.max(sc, axis=-1, keepdims=True)
        p = jnp.exp(sc - m)
        p = p / jnp.sum(p, axis=-1, keepdims=True)
        o_heads.append(dots(p, vi))
    o = jnp.concatenate(o_heads, axis=-1)
    a = dots(o, Wo_ref[l]) + bo_ref[l]
    h = _ln(h + a, ln1_g_ref[l], ln1_b_ref[l])
    f = jnp.zeros((SEG, F), jnp.float32)
    for c in range(DFF // DFF_CHUNK):
        w1c = Wff1_ref[l, :, c * DFF_CHUNK:(c + 1) * DFF_CHUNK]
        b1c = bff1_ref[l, c * DFF_CHUNK:(c + 1) * DFF_CHUNK]
        w2c = Wff2_ref[l, c * DFF_CHUNK:(c + 1) * DFF_CHUNK, :]
        f = f + dots(jnp.maximum(dots(h, w1c) + b1c, 0.0), w2c)
    h = _ln(h + f + bff2_ref[l], ln2_g_ref[l], ln2_b_ref[l])

    l = 1
    k = dots(h, Wk_ref[l]) + bk_ref[l]
    v = dots(h, Wv_ref[l]) + bv_ref[l]
    hl = h[SEG - 1:SEG, :]
    q = dots(hl, Wq_ref[l]) + bq_ref[l]
    o_heads = []
    for i in range(NH):
        qi = q[:, i * DH:(i + 1) * DH]
        ki = k[:, i * DH:(i + 1) * DH]
        vi = v[:, i * DH:(i + 1) * DH]
        sc = dots_t(qi, ki) * inv_sqrt_dh
        m = jnp.max(sc, axis=-1, keepdims=True)
        p = jnp.exp(sc - m)
        p = p / jnp.sum(p, axis=-1, keepdims=True)
        o_heads.append(dots(p, vi))
    o = jnp.concatenate(o_heads, axis=-1)
    a = dots(o, Wo_ref[l]) + bo_ref[l]
    hl = _ln(hl + a, ln1_g_ref[l], ln1_b_ref[l])
    f = jnp.maximum(dots(hl, Wff1_ref[l]) + bff1_ref[l], 0.0)
    f = dots(f, Wff2_ref[l]) + bff2_ref[l]
    hl = _ln(hl + f, ln2_g_ref[l], ln2_b_ref[l])
    out_ref[0] = hl


def _transformer(hb, Wq, Wk, Wv, Wo, bq, bk, bv, bo, ln1_g, ln1_b, ln2_g,
                 ln2_b, Wff1, bff1, Wff2, bff2):
    full = lambda *shape: pl.BlockSpec(shape, lambda i: (0,) * len(shape))
    return pl.pallas_call(
        _transformer_body,
        grid=(NB,),
        in_specs=[
            pl.BlockSpec((1, SEG, F), lambda i: (i, 0, 0)),
            full(NL, F, F), full(NL, F, F), full(NL, F, F), full(NL, F, F),
            full(NL, F), full(NL, F), full(NL, F), full(NL, F),
            full(NL, F), full(NL, F), full(NL, F), full(NL, F),
            full(NL, F, DFF), full(NL, DFF), full(NL, DFF, F), full(NL, F),
        ],
        out_specs=pl.BlockSpec((1, 1, F), lambda i: (i, 0, 0)),
        out_shape=jax.ShapeDtypeStruct((NB, 1, F), jnp.float32),
    )(hb, Wq, Wk, Wv, Wo, bq, bk, bv, bo, ln1_g, ln1_b, ln2_g, ln2_b,
      Wff1, bff1, Wff2, bff2).reshape(NB, F)


def kernel(x, edge_index, batch_num_nodes, W_gat, a_src, a_dst, b_gat, Wq, Wk,
           Wv, Wo, bq, bk, bv, bo, ln1_g, ln1_b, ln2_g, ln2_b, Wff1, bff1,
           Wff2, bff2):
    h = x @ W_gat
    src = edge_index[0]
    dst = edge_index[1]
    e = (h @ a_src)[src] + (h @ a_dst)[dst]
    e = jnp.where(e >= 0, e, 0.2 * e)
    w = jnp.exp(e)
    den = jax.ops.segment_sum(w, dst, num_segments=N_NODES)
    alpha = w / (den[dst] + 1e-16)
    xg = jax.ops.segment_sum(h[src] * alpha[:, None], dst,
                             num_segments=N_NODES)
    xg = xg + b_gat

    hb = xg.reshape(NB, SEG, F)
    return _transformer(hb, Wq, Wk, Wv, Wo, bq, bk, bv, bo, ln1_g, ln1_b,
                        ln2_g, ln2_b, Wff1, bff1, Wff2, bff2)


# speedup vs baseline: 15.7523x; 13.0961x over previous
import functools
import math

import jax
import jax.numpy as jnp
from jax import lax
from jax.experimental import pallas as pl
from jax.experimental.pallas import tpu as pltpu
from jax.experimental.pallas import tpu_sc as plsc

N_NODES = 10000
N_EDGES = 320000
F = 128
NB = 16
SEG = 625
NL = 2
NH = 4
DH = F // NH
DFF = 2048
DFF_CHUNK = 512

NC = 2
NS = 16
NW = NC * NS
EPT = N_EDGES // NW
CHUNK = 80
NCHUNK = EPT // CHUNK
N_PAD = 10240
ROWS_PT = N_PAD // NS
NSUP = 5
CPS = NCHUNK // NSUP

_dots = functools.partial(lax.dot_general,
                          dimension_numbers=(((1,), (0,)), ((), ())),
                          preferred_element_type=jnp.float32)
_dots_t = functools.partial(lax.dot_general,
                            dimension_numbers=(((1,), (1,)), ((), ())),
                            preferred_element_type=jnp.float32)


def _proj_body(x_ref, W_ref, asrc_ref, adst_ref, h_ref, es_ref, ed_ref):
    h = _dots(x_ref[0], W_ref[...])
    h_ref[0] = h
    es_ref[0] = _dots_t(h, asrc_ref[...])
    ed_ref[0] = _dots_t(h, adst_ref[...])


def _projections(x, W_gat, a_src, a_dst):
    h, es, ed = pl.pallas_call(
        _proj_body,
        grid=(NB,),
        in_specs=[
            pl.BlockSpec((1, SEG, F), lambda i: (i, 0, 0)),
            pl.BlockSpec((F, F), lambda i: (0, 0)),
            pl.BlockSpec((1, F), lambda i: (0, 0)),
            pl.BlockSpec((1, F), lambda i: (0, 0)),
        ],
        out_specs=[
            pl.BlockSpec((1, SEG, F), lambda i: (i, 0, 0)),
            pl.BlockSpec((1, SEG, 1), lambda i: (i, 0, 0)),
            pl.BlockSpec((1, SEG, 1), lambda i: (i, 0, 0)),
        ],
        out_shape=[
            jax.ShapeDtypeStruct((NB, SEG, F), jnp.float32),
            jax.ShapeDtypeStruct((NB, SEG, 1), jnp.float32),
            jax.ShapeDtypeStruct((NB, SEG, 1), jnp.float32),
        ],
    )(x.reshape(NB, SEG, F), W_gat, a_src.reshape(1, F),
      a_dst.reshape(1, F))
    return (h.reshape(N_NODES, F), es.reshape(N_NODES), ed.reshape(N_NODES))


def _gat_edges_body(h_hbm, es_hbm, ed_hbm, src2_hbm, dst2_hbm,
                    den_hbm, acc_hbm,
                    src_v, dst_v, es_c, ed_c, den_v, rows_v, rows2_v,
                    acc_sh, sem, sem2):
    cid = lax.axis_index("c")
    sid = lax.axis_index("s")
    wid = cid * NS + sid

    def zero_row(r, _):
        for k in range(8):
            rows_v[r, pl.ds(k * 16, 16)] = jnp.zeros((16,), jnp.float32)
        return 0
    lax.fori_loop(0, CHUNK, zero_row, 0)
    for j in range(ROWS_PT // CHUNK):
        pltpu.sync_copy(rows_v,
                        acc_sh.at[pl.ds(sid * ROWS_PT + j * CHUNK, CHUNK)])

    def zero_den(i, _):
        den_v[pl.ds(i * 16, 16)] = jnp.zeros((16,), jnp.float32)
        return 0
    lax.fori_loop(0, N_NODES // 16, zero_den, 0)

    plsc.subcore_barrier()

    def super_body(s, _):
        pltpu.sync_copy(src2_hbm.at[wid, s], src_v)
        pltpu.sync_copy(dst2_hbm.at[wid, s], dst_v)

        def chunk_body(c, _):
            cp_es = pltpu.async_copy(es_hbm.at[src_v.at[c]], es_c, sem)
            cp_ed = pltpu.async_copy(ed_hbm.at[dst_v.at[c]], ed_c, sem)
            cp_h = pltpu.async_copy(h_hbm.at[src_v.at[c]], rows_v, sem2)
            cp_es.wait()
            cp_ed.wait()
            cp_h.wait()
            for g in range(CHUNK // 16):
                sl = pl.ds(g * 16, 16)
                e = es_c[sl] + ed_c[sl]
                e = jnp.where(e >= 0.0, e, 0.2 * e)
                w = jnp.exp(e)
                plsc.addupdate_scatter(den_v, [dst_v[c, sl]], w)
                for r in range(16):
                    b16 = jnp.take_along_axis(
                        w, jnp.full((16,), r, jnp.int32), axis=0)
                    row = g * 16 + r
                    for k in range(8):
                        rows2_v[row, pl.ds(k * 16, 16)] = (
                            rows_v[row, pl.ds(k * 16, 16)] * b16)
            pltpu.sync_copy(rows2_v, acc_sh.at[dst_v.at[c]], add=True)
            return 0
        lax.fori_loop(0, CPS, chunk_body, 0)
        return 0
    lax.fori_loop(0, NSUP, super_body, 0)
    pltpu.sync_copy(den_v, den_hbm.at[pl.ds(wid * N_NODES, N_NODES)])

    plsc.subcore_barrier()
    pltpu.sync_copy(
        acc_sh.at[pl.ds(sid * ROWS_PT, ROWS_PT)],
        acc_hbm.at[pl.ds((cid * N_PAD + sid * ROWS_PT), ROWS_PT)])


def _gat_edges(h, es, ed, src, dst):
    f = pl.kernel(
        _gat_edges_body,
        out_type=[
            jax.ShapeDtypeStruct((NW * N_NODES,), jnp.float32),
            jax.ShapeDtypeStruct((NC * N_PAD, F), jnp.float32),
        ],
        mesh=plsc.VectorSubcoreMesh(core_axis_name="c",
                                    subcore_axis_name="s"),
        compiler_params=pltpu.CompilerParams(needs_layout_passes=False),
        scratch_types=[
            pltpu.VMEM((CPS, CHUNK), jnp.int32),
            pltpu.VMEM((CPS, CHUNK), jnp.int32),
            pltpu.VMEM((CHUNK,), jnp.float32),
            pltpu.VMEM((CHUNK,), jnp.float32),
            pltpu.VMEM((N_NODES,), jnp.float32),
            pltpu.VMEM((CHUNK, F), jnp.float32),
            pltpu.VMEM((CHUNK, F), jnp.float32),
            pltpu.VMEM_SHARED((N_PAD, F), jnp.float32),
            pltpu.SemaphoreType.DMA,
            pltpu.SemaphoreType.DMA,
        ],
    )
    den, acc = f(h, es, ed, src.reshape(NW, NSUP, CPS, CHUNK),
                 dst.reshape(NW, NSUP, CPS, CHUNK))
    acc = acc.reshape(NC, N_PAD, F)[:, :N_NODES]
    return (den.reshape(NW, NB, SEG, 1),
            acc.reshape(NC, NB, SEG, F))


def _ln(h, g, b, eps=1e-5):
    mu = jnp.mean(h, axis=-1, keepdims=True)
    var = jnp.mean((h - mu) ** 2, axis=-1, keepdims=True)
    return (h - mu) * jax.lax.rsqrt(var + eps) * g + b


def _transformer_body(acc_ref, den_ref, bgat_ref, Wq_ref, Wk_ref, Wv_ref,
                      Wo_ref, bq_ref, bk_ref, bv_ref, bo_ref, ln1_g_ref,
                      ln1_b_ref, ln2_g_ref, ln2_b_ref, Wff1_ref, bff1_ref,
                      Wff2_ref, bff2_ref, out_ref):
    den = jnp.sum(den_ref[:, 0], axis=0)
    num = acc_ref[0, 0] + acc_ref[1, 0]
    h = num * (1.0 / (den + 1e-16)) + bgat_ref[...]
    inv_sqrt_dh = 1.0 / math.sqrt(DH)

    l = 0
    q = _dots(h, Wq_ref[l]) + bq_ref[l]
    k = _dots(h, Wk_ref[l]) + bk_ref[l]
    v = _dots(h, Wv_ref[l]) + bv_ref[l]
    o_heads = []
    for i in range(NH):
        qi = q[:, i * DH:(i + 1) * DH]
        ki = k[:, i * DH:(i + 1) * DH]
        vi = v[:, i * DH:(i + 1) * DH]
        sc = _dots_t(qi, ki) * inv_sqrt_dh
        m = jnp.max(sc, axis=-1, keepdims=True)
        p = jnp.exp(sc - m)
        p = p / jnp.sum(p, axis=-1, keepdims=True)
        o_heads.append(_dots(p, vi))
    o = jnp.concatenate(o_heads, axis=-1)
    a = _dots(o, Wo_ref[l]) + bo_ref[l]
    h = _ln(h + a, ln1_g_ref[l], ln1_b_ref[l])
    f = jnp.zeros((SEG, F), jnp.float32)
    for c in range(DFF // DFF_CHUNK):
        w1c = Wff1_ref[l, :, c * DFF_CHUNK:(c + 1) * DFF_CHUNK]
        b1c = bff1_ref[l, c * DFF_CHUNK:(c + 1) * DFF_CHUNK]
        w2c = Wff2_ref[l, c * DFF_CHUNK:(c + 1) * DFF_CHUNK, :]
        f = f + _dots(jnp.maximum(_dots(h, w1c) + b1c, 0.0), w2c)
    h = _ln(h + f + bff2_ref[l], ln2_g_ref[l], ln2_b_ref[l])

    l = 1
    k = _dots(h, Wk_ref[l]) + bk_ref[l]
    v = _dots(h, Wv_ref[l]) + bv_ref[l]
    hl = h[SEG - 1:SEG, :]
    q = _dots(hl, Wq_ref[l]) + bq_ref[l]
    o_heads = []
    for i in range(NH):
        qi = q[:, i * DH:(i + 1) * DH]
        ki = k[:, i * DH:(i + 1) * DH]
        vi = v[:, i * DH:(i + 1) * DH]
        sc = _dots_t(qi, ki) * inv_sqrt_dh
        m = jnp.max(sc, axis=-1, keepdims=True)
        p = jnp.exp(sc - m)
        p = p / jnp.sum(p, axis=-1, keepdims=True)
        o_heads.append(_dots(p, vi))
    o = jnp.concatenate(o_heads, axis=-1)
    a = _dots(o, Wo_ref[l]) + bo_ref[l]
    hl = _ln(hl + a, ln1_g_ref[l], ln1_b_ref[l])
    f = jnp.maximum(_dots(hl, Wff1_ref[l]) + bff1_ref[l], 0.0)
    f = _dots(f, Wff2_ref[l]) + bff2_ref[l]
    hl = _ln(hl + f, ln2_g_ref[l], ln2_b_ref[l])
    out_ref[0] = hl


def _transformer(acc, den, b_gat, Wq, Wk, Wv, Wo, bq, bk, bv, bo, ln1_g,
                 ln1_b, ln2_g, ln2_b, Wff1, bff1, Wff2, bff2):
    full = lambda *shape: pl.BlockSpec(shape, lambda i: (0,) * len(shape))
    return pl.pallas_call(
        _transformer_body,
        grid=(NB,),
        in_specs=[
            pl.BlockSpec((NC, 1, SEG, F), lambda i: (0, i, 0, 0)),
            pl.BlockSpec((NW, 1, SEG, 1), lambda i: (0, i, 0, 0)),
            full(1, F),
            full(NL, F, F), full(NL, F, F), full(NL, F, F), full(NL, F, F),
            full(NL, F), full(NL, F), full(NL, F), full(NL, F),
            full(NL, F), full(NL, F), full(NL, F), full(NL, F),
            full(NL, F, DFF), full(NL, DFF), full(NL, DFF, F), full(NL, F),
        ],
        out_specs=pl.BlockSpec((1, 1, F), lambda i: (i, 0, 0)),
        out_shape=jax.ShapeDtypeStruct((NB, 1, F), jnp.float32),
    )(acc, den, b_gat.reshape(1, F), Wq, Wk, Wv, Wo, bq, bk, bv, bo,
      ln1_g, ln1_b, ln2_g, ln2_b, Wff1, bff1, Wff2,
      bff2).reshape(NB, F)


def kernel(x, edge_index, batch_num_nodes, W_gat, a_src, a_dst, b_gat, Wq, Wk,
           Wv, Wo, bq, bk, bv, bo, ln1_g, ln1_b, ln2_g, ln2_b, Wff1, bff1,
           Wff2, bff2):
    h, es, ed = _projections(x, W_gat, a_src, a_dst)
    src = edge_index[0].astype(jnp.int32)
    dst = edge_index[1].astype(jnp.int32)
    den, acc = _gat_edges(h, es, ed, src, dst)
    return _transformer(acc, den, b_gat, Wq, Wk, Wv, Wo, bq, bk, bv, bo,
                        ln1_g, ln1_b, ln2_g, ln2_b, Wff1, bff1, Wff2, bff2)


# speedup vs baseline: 17.0614x; 1.0831x over previous
import functools
import math

import jax
import jax.numpy as jnp
from jax import lax
from jax.experimental import pallas as pl
from jax.experimental.pallas import tpu as pltpu
from jax.experimental.pallas import tpu_sc as plsc

N_NODES = 10000
N_EDGES = 320000
F = 128
NB = 16
SEG = 625
NL = 2
NH = 4
DH = F // NH
DFF = 2048
DFF_CHUNK = 512

NC = 2
NS = 16
NW = NC * NS
EPT = N_EDGES // NW
CHUNK = 80
NCHUNK = EPT // CHUNK
N_PAD = 10240
ROWS_PT = N_PAD // NS
NSUP = 5
CPS = NCHUNK // NSUP

_dots = functools.partial(lax.dot_general,
                          dimension_numbers=(((1,), (0,)), ((), ())),
                          preferred_element_type=jnp.float32)
_dots_t = functools.partial(lax.dot_general,
                            dimension_numbers=(((1,), (1,)), ((), ())),
                            preferred_element_type=jnp.float32)


def _proj_body(x_ref, W_ref, asrc_ref, adst_ref, h_ref, es_ref, ed_ref):
    h = _dots(x_ref[0], W_ref[...])
    h_ref[0] = h
    es_ref[0] = _dots_t(h, asrc_ref[...])
    ed_ref[0] = _dots_t(h, adst_ref[...])


def _projections(x, W_gat, a_src, a_dst):
    h, es, ed = pl.pallas_call(
        _proj_body,
        grid=(NB,),
        in_specs=[
            pl.BlockSpec((1, SEG, F), lambda i: (i, 0, 0)),
            pl.BlockSpec((F, F), lambda i: (0, 0)),
            pl.BlockSpec((1, F), lambda i: (0, 0)),
            pl.BlockSpec((1, F), lambda i: (0, 0)),
        ],
        out_specs=[
            pl.BlockSpec((1, SEG, F), lambda i: (i, 0, 0)),
            pl.BlockSpec((1, SEG, 1), lambda i: (i, 0, 0)),
            pl.BlockSpec((1, SEG, 1), lambda i: (i, 0, 0)),
        ],
        out_shape=[
            jax.ShapeDtypeStruct((NB, SEG, F), jnp.float32),
            jax.ShapeDtypeStruct((NB, SEG, 1), jnp.float32),
            jax.ShapeDtypeStruct((NB, SEG, 1), jnp.float32),
        ],
    )(x.reshape(NB, SEG, F), W_gat, a_src.reshape(1, F),
      a_dst.reshape(1, F))
    return (h.reshape(N_NODES, F), es.reshape(N_NODES), ed.reshape(N_NODES))


def _gat_edges_body(h_hbm, es_hbm, ed_hbm, src2_hbm, dst2_hbm,
                    den_hbm, acc_hbm,
                    src_v, dst_v, es_c, ed_c, es2_c, ed2_c, den_v, rows_v,
                    rows2_v, acc_sh, sem, sem2):
    cid = lax.axis_index("c")
    sid = lax.axis_index("s")
    wid = cid * NS + sid

    def zero_row(r, _):
        for k in range(8):
            rows_v[r, pl.ds(k * 16, 16)] = jnp.zeros((16,), jnp.float32)
        return 0
    lax.fori_loop(0, CHUNK, zero_row, 0)
    for j in range(ROWS_PT // CHUNK):
        pltpu.sync_copy(rows_v,
                        acc_sh.at[pl.ds(sid * ROWS_PT + j * CHUNK, CHUNK)])

    def zero_den(i, _):
        den_v[pl.ds(i * 16, 16)] = jnp.zeros((16,), jnp.float32)
        return 0
    lax.fori_loop(0, N_NODES // 16, zero_den, 0)

    plsc.subcore_barrier()

    bufs = ((es_c, ed_c, rows_v, sem), (es2_c, ed2_c, rows2_v, sem2))

    def start(c, b):
        es_b, ed_b, rows_b, sem_b = bufs[b]
        pltpu.async_copy(es_hbm.at[src_v.at[c]], es_b, sem_b)
        pltpu.async_copy(ed_hbm.at[dst_v.at[c]], ed_b, sem_b)
        pltpu.async_copy(h_hbm.at[src_v.at[c]], rows_b, sem_b)

    def finish(c, b):
        es_b, ed_b, rows_b, sem_b = bufs[b]
        pltpu.make_async_copy(es_hbm.at[src_v.at[c]], es_b, sem_b).wait()
        pltpu.make_async_copy(ed_hbm.at[dst_v.at[c]], ed_b, sem_b).wait()
        pltpu.make_async_copy(h_hbm.at[src_v.at[c]], rows_b, sem_b).wait()
        for g in range(CHUNK // 16):
            sl = pl.ds(g * 16, 16)
            e = es_b[sl] + ed_b[sl]
            e = jnp.where(e >= 0.0, e, 0.2 * e)
            w = jnp.exp(e)
            plsc.addupdate_scatter(den_v, [dst_v[c, sl]], w)
            for r in range(16):
                b16 = jnp.take_along_axis(
                    w, jnp.full((16,), r, jnp.int32), axis=0)
                row = g * 16 + r
                for k in range(8):
                    rows_b[row, pl.ds(k * 16, 16)] = (
                        rows_b[row, pl.ds(k * 16, 16)] * b16)
        pltpu.sync_copy(rows_b, acc_sh.at[dst_v.at[c]], add=True)

    def super_body(s, _):
        pltpu.sync_copy(src2_hbm.at[wid, s], src_v)
        pltpu.sync_copy(dst2_hbm.at[wid, s], dst_v)
        start(0, 0)

        def pair_body(i, _):
            start(2 * i + 1, 1)
            finish(2 * i, 0)
            start(2 * i + 2, 0)
            finish(2 * i + 1, 1)
            return 0
        lax.fori_loop(0, (CPS - 1) // 2, pair_body, 0)
        finish(CPS - 1, 0)
        return 0
    lax.fori_loop(0, NSUP, super_body, 0)
    pltpu.sync_copy(den_v, den_hbm.at[pl.ds(wid * N_NODES, N_NODES)])

    plsc.subcore_barrier()
    pltpu.sync_copy(
        acc_sh.at[pl.ds(sid * ROWS_PT, ROWS_PT)],
        acc_hbm.at[pl.ds((cid * N_PAD + sid * ROWS_PT), ROWS_PT)])


def _gat_edges(h, es, ed, src, dst):
    f = pl.kernel(
        _gat_edges_body,
        out_type=[
            jax.ShapeDtypeStruct((NW * N_NODES,), jnp.float32),
            jax.ShapeDtypeStruct((NC * N_PAD, F), jnp.float32),
        ],
        mesh=plsc.VectorSubcoreMesh(core_axis_name="c",
                                    subcore_axis_name="s"),
        compiler_params=pltpu.CompilerParams(needs_layout_passes=False),
        scratch_types=[
            pltpu.VMEM((CPS, CHUNK), jnp.int32),
            pltpu.VMEM((CPS, CHUNK), jnp.int32),
            pltpu.VMEM((CHUNK,), jnp.float32),
            pltpu.VMEM((CHUNK,), jnp.float32),
            pltpu.VMEM((CHUNK,), jnp.float32),
            pltpu.VMEM((CHUNK,), jnp.float32),
            pltpu.VMEM((N_NODES,), jnp.float32),
            pltpu.VMEM((CHUNK, F), jnp.float32),
            pltpu.VMEM((CHUNK, F), jnp.float32),
            pltpu.VMEM_SHARED((N_PAD, F), jnp.float32),
            pltpu.SemaphoreType.DMA,
            pltpu.SemaphoreType.DMA,
        ],
    )
    den, acc = f(h, es, ed, src.reshape(NW, NSUP, CPS, CHUNK),
                 dst.reshape(NW, NSUP, CPS, CHUNK))
    acc = acc.reshape(NC, N_PAD, F)[:, :N_NODES]
    return (den.reshape(NW, NB, SEG, 1),
            acc.reshape(NC, NB, SEG, F))


def _ln(h, g, b, eps=1e-5):
    mu = jnp.mean(h, axis=-1, keepdims=True)
    var = jnp.mean((h - mu) ** 2, axis=-1, keepdims=True)
    return (h - mu) * jax.lax.rsqrt(var + eps) * g + b


def _transformer_body(acc_ref, den_ref, bgat_ref, Wq_ref, Wk_ref, Wv_ref,
                      Wo_ref, bq_ref, bk_ref, bv_ref, bo_ref, ln1_g_ref,
                      ln1_b_ref, ln2_g_ref, ln2_b_ref, Wff1_ref, bff1_ref,
                      Wff2_ref, bff2_ref, out_ref):
    den = jnp.sum(den_ref[:, 0], axis=0)
    num = acc_ref[0, 0] + acc_ref[1, 0]
    h = num * (1.0 / (den + 1e-16)) + bgat_ref[...]
    inv_sqrt_dh = 1.0 / math.sqrt(DH)

    l = 0
    q = _dots(h, Wq_ref[l]) + bq_ref[l]
    k = _dots(h, Wk_ref[l]) + bk_ref[l]
    v = _dots(h, Wv_ref[l]) + bv_ref[l]
    o_heads = []
    for i in range(NH):
        qi = q[:, i * DH:(i + 1) * DH]
        ki = k[:, i * DH:(i + 1) * DH]
        vi = v[:, i * DH:(i + 1) * DH]
        sc = _dots_t(qi, ki) * inv_sqrt_dh
        m = jnp.max(sc, axis=-1, keepdims=True)
        p = jnp.exp(sc - m)
        p = p / jnp.sum(p, axis=-1, keepdims=True)
        o_heads.append(_dots(p, vi))
    o = jnp.concatenate(o_heads, axis=-1)
    a = _dots(o, Wo_ref[l]) + bo_ref[l]
    h = _ln(h + a, ln1_g_ref[l], ln1_b_ref[l])
    f = jnp.zeros((SEG, F), jnp.float32)
    for c in range(DFF // DFF_CHUNK):
        w1c = Wff1_ref[l, :, c * DFF_CHUNK:(c + 1) * DFF_CHUNK]
        b1c = bff1_ref[l, c * DFF_CHUNK:(c + 1) * DFF_CHUNK]
        w2c = Wff2_ref[l, c * DFF_CHUNK:(c + 1) * DFF_CHUNK, :]
        f = f + _dots(jnp.maximum(_dots(h, w1c) + b1c, 0.0), w2c)
    h = _ln(h + f + bff2_ref[l], ln2_g_ref[l], ln2_b_ref[l])

    l = 1
    k = _dots(h, Wk_ref[l]) + bk_ref[l]
    v = _dots(h, Wv_ref[l]) + bv_ref[l]
    hl = h[SEG - 1:SEG, :]
    q = _dots(hl, Wq_ref[l]) + bq_ref[l]
    o_heads = []
    for i in range(NH):
        qi = q[:, i * DH:(i + 1) * DH]
        ki = k[:, i * DH:(i + 1) * DH]
        vi = v[:, i * DH:(i + 1) * DH]
        sc = _dots_t(qi, ki) * inv_sqrt_dh
        m = jnp.max(sc, axis=-1, keepdims=True)
        p = jnp.exp(sc - m)
        p = p / jnp.sum(p, axis=-1, keepdims=True)
        o_heads.append(_dots(p, vi))
    o = jnp.concatenate(o_heads, axis=-1)
    a = _dots(o, Wo_ref[l]) + bo_ref[l]
    hl = _ln(hl + a, ln1_g_ref[l], ln1_b_ref[l])
    f = jnp.maximum(_dots(hl, Wff1_ref[l]) + bff1_ref[l], 0.0)
    f = _dots(f, Wff2_ref[l]) + bff2_ref[l]
    hl = _ln(hl + f, ln2_g_ref[l], ln2_b_ref[l])
    out_ref[0] = hl


def _transformer(acc, den, b_gat, Wq, Wk, Wv, Wo, bq, bk, bv, bo, ln1_g,
                 ln1_b, ln2_g, ln2_b, Wff1, bff1, Wff2, bff2):
    full = lambda *shape: pl.BlockSpec(shape, lambda i: (0,) * len(shape))
    return pl.pallas_call(
        _transformer_body,
        grid=(NB,),
        in_specs=[
            pl.BlockSpec((NC, 1, SEG, F), lambda i: (0, i, 0, 0)),
            pl.BlockSpec((NW, 1, SEG, 1), lambda i: (0, i, 0, 0)),
            full(1, F),
            full(NL, F, F), full(NL, F, F), full(NL, F, F), full(NL, F, F),
            full(NL, F), full(NL, F), full(NL, F), full(NL, F),
            full(NL, F), full(NL, F), full(NL, F), full(NL, F),
            full(NL, F, DFF), full(NL, DFF), full(NL, DFF, F), full(NL, F),
        ],
        out_specs=pl.BlockSpec((1, 1, F), lambda i: (i, 0, 0)),
        out_shape=jax.ShapeDtypeStruct((NB, 1, F), jnp.float32),
    )(acc, den, b_gat.reshape(1, F), Wq, Wk, Wv, Wo, bq, bk, bv, bo,
      ln1_g, ln1_b, ln2_g, ln2_b, Wff1, bff1, Wff2,
      bff2).reshape(NB, F)


def kernel(x, edge_index, batch_num_nodes, W_gat, a_src, a_dst, b_gat, Wq, Wk,
           Wv, Wo, bq, bk, bv, bo, ln1_g, ln1_b, ln2_g, ln2_b, Wff1, bff1,
           Wff2, bff2):
    h, es, ed = _projections(x, W_gat, a_src, a_dst)
    src = edge_index[0].astype(jnp.int32)
    dst = edge_index[1].astype(jnp.int32)
    den, acc = _gat_edges(h, es, ed, src, dst)
    return _transformer(acc, den, b_gat, Wq, Wk, Wv, Wo, bq, bk, bv, bo,
                        ln1_g, ln1_b, ln2_g, ln2_b, Wff1, bff1, Wff2, bff2)


# speedup vs baseline: 17.7111x; 1.0381x over previous
import functools
import math

import jax
import jax.numpy as jnp
from jax import lax
from jax.experimental import pallas as pl
from jax.experimental.pallas import tpu as pltpu
from jax.experimental.pallas import tpu_sc as plsc

N_NODES = 10000
N_EDGES = 320000
F = 128
NB = 16
SEG = 625
NL = 2
NH = 4
DH = F // NH
DFF = 2048
DFF_CHUNK = 512

NC = 2
NS = 16
NW = NC * NS
EPT = N_EDGES // NW
CHUNK = 80
NCHUNK = EPT // CHUNK
N_PAD = 10240
SEGP = N_PAD // NB
ROWS_PT = N_PAD // NS
NSUP = 5
CPS = NCHUNK // NSUP

_dots = functools.partial(lax.dot_general,
                          dimension_numbers=(((1,), (0,)), ((), ())),
                          preferred_element_type=jnp.float32)
_dots_t = functools.partial(lax.dot_general,
                            dimension_numbers=(((1,), (1,)), ((), ())),
                            preferred_element_type=jnp.float32)


def _proj_body(x_ref, W_ref, asrc_ref, adst_ref, h_ref, es_ref, ed_ref):
    h = _dots(x_ref[0], W_ref[...])
    h_ref[0] = h
    es_ref[0] = _dots_t(h, asrc_ref[...])
    ed_ref[0] = _dots_t(h, adst_ref[...])


def _projections(x, W_gat, a_src, a_dst):
    h, es, ed = pl.pallas_call(
        _proj_body,
        grid=(NB,),
        in_specs=[
            pl.BlockSpec((1, SEG, F), lambda i: (i, 0, 0)),
            pl.BlockSpec((F, F), lambda i: (0, 0)),
            pl.BlockSpec((1, F), lambda i: (0, 0)),
            pl.BlockSpec((1, F), lambda i: (0, 0)),
        ],
        out_specs=[
            pl.BlockSpec((1, SEG, F), lambda i: (i, 0, 0)),
            pl.BlockSpec((1, SEG, 1), lambda i: (i, 0, 0)),
            pl.BlockSpec((1, SEG, 1), lambda i: (i, 0, 0)),
        ],
        out_shape=[
            jax.ShapeDtypeStruct((NB, SEG, F), jnp.float32),
            jax.ShapeDtypeStruct((NB, SEG, 1), jnp.float32),
            jax.ShapeDtypeStruct((NB, SEG, 1), jnp.float32),
        ],
    )(x.reshape(NB, SEG, F), W_gat, a_src.reshape(1, F),
      a_dst.reshape(1, F))
    return (h.reshape(N_NODES, F), es.reshape(N_NODES), ed.reshape(N_NODES))


def _gat_edges_body(h_hbm, es_hbm, ed_hbm, src2_hbm, dst2_hbm,
                    den_hbm, acc_hbm,
                    src_v, dst_v, es_c, ed_c, es2_c, ed2_c, den_v, rows_v,
                    rows2_v, acc_sh, sem, sem2):
    cid = lax.axis_index("c")
    sid = lax.axis_index("s")
    wid = cid * NS + sid

    def zero_row(r, _):
        for k in range(8):
            rows_v[r, pl.ds(k * 16, 16)] = jnp.zeros((16,), jnp.float32)
        return 0
    lax.fori_loop(0, CHUNK, zero_row, 0)
    for j in range(ROWS_PT // CHUNK):
        pltpu.sync_copy(rows_v,
                        acc_sh.at[pl.ds(sid * ROWS_PT + j * CHUNK, CHUNK)])

    def zero_den(i, _):
        den_v[pl.ds(i * 16, 16)] = jnp.zeros((16,), jnp.float32)
        return 0
    lax.fori_loop(0, N_PAD // 16, zero_den, 0)

    plsc.subcore_barrier()

    bufs = ((es_c, ed_c, rows_v, sem), (es2_c, ed2_c, rows2_v, sem2))

    def start(c, b):
        es_b, ed_b, rows_b, sem_b = bufs[b]
        pltpu.async_copy(es_hbm.at[src_v.at[c]], es_b, sem_b)
        pltpu.async_copy(ed_hbm.at[dst_v.at[c]], ed_b, sem_b)
        pltpu.async_copy(h_hbm.at[src_v.at[c]], rows_b, sem_b)

    def finish(c, b):
        es_b, ed_b, rows_b, sem_b = bufs[b]
        pltpu.make_async_copy(es_hbm.at[src_v.at[c]], es_b, sem_b).wait()
        pltpu.make_async_copy(ed_hbm.at[dst_v.at[c]], ed_b, sem_b).wait()
        pltpu.make_async_copy(h_hbm.at[src_v.at[c]], rows_b, sem_b).wait()
        for g in range(CHUNK // 16):
            sl = pl.ds(g * 16, 16)
            e = es_b[sl] + ed_b[sl]
            e = jnp.where(e >= 0.0, e, 0.2 * e)
            w = jnp.exp(e)
            plsc.addupdate_scatter(den_v, [dst_v[c, sl]], w)
            for r in range(16):
                b16 = jnp.take_along_axis(
                    w, jnp.full((16,), r, jnp.int32), axis=0)
                row = g * 16 + r
                for k in range(8):
                    rows_b[row, pl.ds(k * 16, 16)] = (
                        rows_b[row, pl.ds(k * 16, 16)] * b16)
        pltpu.sync_copy(rows_b, acc_sh.at[dst_v.at[c]], add=True)

    def super_body(s, _):
        pltpu.sync_copy(src2_hbm.at[wid, s], src_v)
        pltpu.sync_copy(dst2_hbm.at[wid, s], dst_v)
        start(0, 0)

        def pair_body(i, _):
            start(2 * i + 1, 1)
            finish(2 * i, 0)
            start(2 * i + 2, 0)
            finish(2 * i + 1, 1)
            return 0
        lax.fori_loop(0, (CPS - 1) // 2, pair_body, 0)
        finish(CPS - 1, 0)
        return 0
    lax.fori_loop(0, NSUP, super_body, 0)
    pltpu.sync_copy(den_v, den_hbm.at[pl.ds(wid * N_PAD, N_PAD)])

    plsc.subcore_barrier()
    pltpu.sync_copy(
        acc_sh.at[pl.ds(sid * ROWS_PT, ROWS_PT)],
        acc_hbm.at[pl.ds((cid * N_PAD + sid * ROWS_PT), ROWS_PT)])


def _gat_edges(h, es, ed, src, dst):
    f = pl.kernel(
        _gat_edges_body,
        out_type=[
            jax.ShapeDtypeStruct((NW * N_PAD,), jnp.float32),
            jax.ShapeDtypeStruct((NC * N_PAD, F), jnp.float32),
        ],
        mesh=plsc.VectorSubcoreMesh(core_axis_name="c",
                                    subcore_axis_name="s"),
        compiler_params=pltpu.CompilerParams(needs_layout_passes=False),
        scratch_types=[
            pltpu.VMEM((CPS, CHUNK), jnp.int32),
            pltpu.VMEM((CPS, CHUNK), jnp.int32),
            pltpu.VMEM((CHUNK,), jnp.float32),
            pltpu.VMEM((CHUNK,), jnp.float32),
            pltpu.VMEM((CHUNK,), jnp.float32),
            pltpu.VMEM((CHUNK,), jnp.float32),
            pltpu.VMEM((N_PAD,), jnp.float32),
            pltpu.VMEM((CHUNK, F), jnp.float32),
            pltpu.VMEM((CHUNK, F), jnp.float32),
            pltpu.VMEM_SHARED((N_PAD, F), jnp.float32),
            pltpu.SemaphoreType.DMA,
            pltpu.SemaphoreType.DMA,
        ],
    )
    den, acc = f(h, es, ed, src.reshape(NW, NSUP, CPS, CHUNK),
                 dst.reshape(NW, NSUP, CPS, CHUNK))
    return (den.reshape(NW, NB, SEGP, 1),
            acc.reshape(NC, NB, SEGP, F))


def _ln(h, g, b, eps=1e-5):
    mu = jnp.mean(h, axis=-1, keepdims=True)
    var = jnp.mean((h - mu) ** 2, axis=-1, keepdims=True)
    return (h - mu) * jax.lax.rsqrt(var + eps) * g + b


def _transformer_body(acc_ref, den_ref, bgat_ref, Wq_ref, Wk_ref, Wv_ref,
                      Wo_ref, bq_ref, bk_ref, bv_ref, bo_ref, ln1_g_ref,
                      ln1_b_ref, ln2_g_ref, ln2_b_ref, Wff1_ref, bff1_ref,
                      Wff2_ref, bff2_ref, out_ref):
    den = jnp.sum(den_ref[:, 0], axis=0)
    num = acc_ref[0, 0] + acc_ref[1, 0]
    h = num * (1.0 / (den + 1e-16)) + bgat_ref[...]
    inv_sqrt_dh = 1.0 / math.sqrt(DH)
    kmask = (jax.lax.broadcasted_iota(jnp.int32, (1, SEGP), 1)
             < SEG)

    l = 0
    q = _dots(h, Wq_ref[l]) + bq_ref[l]
    k = _dots(h, Wk_ref[l]) + bk_ref[l]
    v = _dots(h, Wv_ref[l]) + bv_ref[l]
    o_heads = []
    for i in range(NH):
        qi = q[:, i * DH:(i + 1) * DH]
        ki = k[:, i * DH:(i + 1) * DH]
        vi = v[:, i * DH:(i + 1) * DH]
        sc = _dots_t(qi, ki) * inv_sqrt_dh
        sc = jnp.where(kmask, sc, -1e30)
        m = jnp.max(sc, axis=-1, keepdims=True)
        p = jnp.exp(sc - m)
        p = p / jnp.sum(p, axis=-1, keepdims=True)
        o_heads.append(_dots(p, vi))
    o = jnp.concatenate(o_heads, axis=-1)
    a = _dots(o, Wo_ref[l]) + bo_ref[l]
    h = _ln(h + a, ln1_g_ref[l], ln1_b_ref[l])
    f = jnp.zeros((SEGP, F), jnp.float32)
    for c in range(DFF // DFF_CHUNK):
        w1c = Wff1_ref[l, :, c * DFF_CHUNK:(c + 1) * DFF_CHUNK]
        b1c = bff1_ref[l, c * DFF_CHUNK:(c + 1) * DFF_CHUNK]
        w2c = Wff2_ref[l, c * DFF_CHUNK:(c + 1) * DFF_CHUNK, :]
        f = f + _dots(jnp.maximum(_dots(h, w1c) + b1c, 0.0), w2c)
    h = _ln(h + f + bff2_ref[l], ln2_g_ref[l], ln2_b_ref[l])

    l = 1
    k = _dots(h, Wk_ref[l]) + bk_ref[l]
    v = _dots(h, Wv_ref[l]) + bv_ref[l]
    hl = h[SEG - 1:SEG, :]
    q = _dots(hl, Wq_ref[l]) + bq_ref[l]
    o_heads = []
    for i in range(NH):
        qi = q[:, i * DH:(i + 1) * DH]
        ki = k[:, i * DH:(i + 1) * DH]
        vi = v[:, i * DH:(i + 1) * DH]
        sc = _dots_t(qi, ki) * inv_sqrt_dh
        sc = jnp.where(kmask, sc, -1e30)
        m = jnp.max(sc, axis=-1, keepdims=True)
        p = jnp.exp(sc - m)
        p = p / jnp.sum(p, axis=-1, keepdims=True)
        o_heads.append(_dots(p, vi))
    o = jnp.concatenate(o_heads, axis=-1)
    a = _dots(o, Wo_ref[l]) + bo_ref[l]
    hl = _ln(hl + a, ln1_g_ref[l], ln1_b_ref[l])
    f = jnp.maximum(_dots(hl, Wff1_ref[l]) + bff1_ref[l], 0.0)
    f = _dots(f, Wff2_ref[l]) + bff2_ref[l]
    hl = _ln(hl + f, ln2_g_ref[l], ln2_b_ref[l])
    out_ref[0] = hl


def _transformer(acc, den, b_gat, Wq, Wk, Wv, Wo, bq, bk, bv, bo, ln1_g,
                 ln1_b, ln2_g, ln2_b, Wff1, bff1, Wff2, bff2):
    full = lambda *shape: pl.BlockSpec(shape, lambda i: (0,) * len(shape))
    return pl.pallas_call(
        _transformer_body,
        grid=(NB,),
        in_specs=[
            pl.BlockSpec((NC, 1, SEGP, F), lambda i: (0, i, 0, 0)),
            pl.BlockSpec((NW, 1, SEGP, 1), lambda i: (0, i, 0, 0)),
            full(1, F),
            full(NL, F, F), full(NL, F, F), full(NL, F, F), full(NL, F, F),
            full(NL, F), full(NL, F), full(NL, F), full(NL, F),
            full(NL, F), full(NL, F), full(NL, F), full(NL, F),
            full(NL, F, DFF), full(NL, DFF), full(NL, DFF, F), full(NL, F),
        ],
        out_specs=pl.BlockSpec((1, 1, F), lambda i: (i, 0, 0)),
        out_shape=jax.ShapeDtypeStruct((NB, 1, F), jnp.float32),
    )(acc, den, b_gat.reshape(1, F), Wq, Wk, Wv, Wo, bq, bk, bv, bo,
      ln1_g, ln1_b, ln2_g, ln2_b, Wff1, bff1, Wff2,
      bff2).reshape(NB, F)


def kernel(x, edge_index, batch_num_nodes, W_gat, a_src, a_dst, b_gat, Wq, Wk,
           Wv, Wo, bq, bk, bv, bo, ln1_g, ln1_b, ln2_g, ln2_b, Wff1, bff1,
           Wff2, bff2):
    h, es, ed = _projections(x, W_gat, a_src, a_dst)
    src = edge_index[0].astype(jnp.int32)
    dst = edge_index[1].astype(jnp.int32)
    dst = dst + (SEGP - SEG) * (dst // SEG)
    ed_pad = jnp.pad(ed.reshape(NB, SEG), ((0, 0), (0, SEGP - SEG)))
    den, acc = _gat_edges(h, es, ed_pad.reshape(N_PAD), src, dst)
    return _transformer(acc, den, b_gat, Wq, Wk, Wv, Wo, bq, bk, bv, bo,
                        ln1_g, ln1_b, ln2_g, ln2_b, Wff1, bff1, Wff2, bff2)
